# Initial kernel scaffold; baseline (speedup 1.0000x reference)
import jax
import jax.numpy as jnp
from jax.experimental import pallas as pl


def kernel(x, phi_x):
    raise NotImplementedError("write your pallas kernel here")


# speedup vs baseline: 739.4974x; 739.4974x over previous
import functools

import jax
import jax.numpy as jnp
from jax import lax
from jax.experimental import pallas as pl
from jax.experimental.pallas import tpu as pltpu
from jax.experimental.pallas import tpu_sc as plsc

N_POINTS = 2000000
N_CTRL = 4096

NC = 2
NS = 16
NW = NC * NS
LANES = 16

CHUNK = 12544
VECS = CHUNK // LANES
NCHUNKS = 5
PER_W = CHUNK * NCHUNKS
PADDED = PER_W * NW


def _sc_body(x_hbm, phi_hbm, out_hbm, phi_v, x_v, out_v):
    ncp = N_CTRL
    dx = 2.0 / (ncp - 3)
    origin = -1.0 - dx

    wid = lax.axis_index("s") * NC + lax.axis_index("c")
    base = wid * PER_W

    pltpu.sync_copy(phi_hbm, phi_v)

    def chunk_body(c, _):
        off = base + c * CHUNK
        pltpu.sync_copy(x_hbm.at[pl.ds(off, CHUNK)], x_v)

        def vec_body(j, _):
            xv = x_v[pl.ds(j * LANES, LANES)]
            t = (xv - origin) - dx
            s = t / dx
            ind = s.astype(jnp.int32)
            u = s - ind.astype(jnp.float32)
            ind = jnp.minimum(jnp.maximum(ind, 0), ncp - 4)

            v = 1.0 - u
            u2 = u * u
            v2 = v * v
            sixth = jnp.float32(1.0 / 6.0)
            w0 = v2 * v * sixth
            w3 = u2 * u * sixth
            w1 = u2 * (0.5 * u - 1.0) + jnp.float32(2.0 / 3.0)
            w2 = 1.0 - (w0 + w1 + w3)

            acc = w0 * plsc.load_gather(phi_v, [ind])
            acc = acc + w1 * plsc.load_gather(phi_v, [ind + 1])
            acc = acc + w2 * plsc.load_gather(phi_v, [ind + 2])
            acc = acc + w3 * plsc.load_gather(phi_v, [ind + 3])
            out_v[pl.ds(j * LANES, LANES)] = acc
            return 0

        lax.fori_loop(0, VECS, vec_body, 0)
        pltpu.sync_copy(out_v, out_hbm.at[pl.ds(off, CHUNK)])
        return 0

    lax.fori_loop(0, NCHUNKS, chunk_body, 0)


@jax.jit
def kernel(x, phi_x):
    xp = jnp.concatenate([x, jnp.zeros((PADDED - N_POINTS,), jnp.float32)])
    mesh = plsc.VectorSubcoreMesh(core_axis_name="c", subcore_axis_name="s")
    run = pl.kernel(
        _sc_body,
        out_type=jax.ShapeDtypeStruct((PADDED,), jnp.float32),
        mesh=mesh,
        scratch_types=[
            pltpu.VMEM((N_CTRL,), jnp.float32),
            pltpu.VMEM((CHUNK,), jnp.float32),
            pltpu.VMEM((CHUNK,), jnp.float32),
        ],
        compiler_params=pltpu.CompilerParams(needs_layout_passes=False),
    )
    out = run(xp, phi_x)
    return out[:N_POINTS]


# speedup vs baseline: 911.1983x; 1.2322x over previous
import jax
import jax.numpy as jnp
from jax import lax
from jax.experimental import pallas as pl
from jax.experimental.pallas import tpu as pltpu
from jax.experimental.pallas import tpu_sc as plsc

N_POINTS = 2000000
N_CTRL = 4096

NC = 2
NS = 16
NW = NC * NS
LANES = 16

VEC_BASE = 3906
CHUNK_V = 651
NCHUNKS = 6
CHUNK_P = CHUNK_V * LANES

DX = 2.0 / (N_CTRL - 3)
ORIGIN = -1.0 - DX
INV_DX = (N_CTRL - 3) / 2.0
SIXTH = 1.0 / 6.0
TWO_THIRDS = 2.0 / 3.0


def _sc_body(x_hbm, phi_hbm, out_hbm, phi_v, x0, x1, o0, o1,
             si0, si1, so0, so1):
    wid = lax.axis_index("s") * NC + lax.axis_index("c")
    base = VEC_BASE * LANES * wid + LANES * jnp.minimum(wid, 8)

    xb = [x0, x1]
    ob = [o0, o1]
    sin = [si0, si1]
    sout = [so0, so1]

    in_descs = [None, None]
    out_descs = [None, None]
    in_descs[0] = pltpu.async_copy(x_hbm.at[pl.ds(base, CHUNK_P)], x0, si0)

    pltpu.sync_copy(phi_hbm, phi_v)

    def spline_vec(xv):
        t = (xv - ORIGIN) - DX
        s = t * INV_DX
        ind = s.astype(jnp.int32)
        u = s - ind.astype(jnp.float32)
        ind = jnp.minimum(jnp.maximum(ind, 0), N_CTRL - 4)
        u2 = u * u
        u3 = u2 * u
        w3 = u3 * SIXTH
        w1 = (0.5 * u3 - u2) + TWO_THIRDS
        w0 = (SIXTH - w3) + 0.5 * (u2 - u)
        w2 = 1.0 - ((w0 + w1) + w3)
        acc = w0 * plsc.load_gather(phi_v, [ind])
        acc = acc + w1 * plsc.load_gather(phi_v, [ind + 1])
        acc = acc + w2 * plsc.load_gather(phi_v, [ind + 2])
        acc = acc + w3 * plsc.load_gather(phi_v, [ind + 3])
        return acc

    for c in range(NCHUNKS):
        b = c & 1
        if c + 1 < NCHUNKS:
            nb = (c + 1) & 1
            in_descs[nb] = pltpu.async_copy(
                x_hbm.at[pl.ds(base + (c + 1) * CHUNK_P, CHUNK_P)],
                xb[nb], sin[nb])
        in_descs[b].wait()
        if c >= 2:
            out_descs[b].wait()

        x_ref = xb[b]
        o_ref = ob[b]

        def vec_body(j, _, x_ref=x_ref, o_ref=o_ref):
            o_ref[pl.ds(j * LANES, LANES)] = spline_vec(
                x_ref[pl.ds(j * LANES, LANES)])
            return 0

        lax.fori_loop(0, CHUNK_V, vec_body, 0)
        out_descs[b] = pltpu.async_copy(
            ob[b], out_hbm.at[pl.ds(base + c * CHUNK_P, CHUNK_P)], sout[b])

    out_descs[0].wait()
    out_descs[1].wait()

    @pl.when(wid < 8)
    def _tail():
        off = base + VEC_BASE * LANES
        pltpu.sync_copy(x_hbm.at[pl.ds(off, LANES)], x0.at[pl.ds(0, LANES)])
        o0[pl.ds(0, LANES)] = spline_vec(x0[pl.ds(0, LANES)])
        pltpu.sync_copy(o0.at[pl.ds(0, LANES)], out_hbm.at[pl.ds(off, LANES)])


@jax.jit
def kernel(x, phi_x):
    mesh = plsc.VectorSubcoreMesh(core_axis_name="c", subcore_axis_name="s")
    run = pl.kernel(
        _sc_body,
        out_type=jax.ShapeDtypeStruct((N_POINTS,), jnp.float32),
        mesh=mesh,
        scratch_types=[
            pltpu.VMEM((N_CTRL,), jnp.float32),
            pltpu.VMEM((CHUNK_P,), jnp.float32),
            pltpu.VMEM((CHUNK_P,), jnp.float32),
            pltpu.VMEM((CHUNK_P,), jnp.float32),
            pltpu.VMEM((CHUNK_P,), jnp.float32),
            pltpu.SemaphoreType.DMA,
            pltpu.SemaphoreType.DMA,
            pltpu.SemaphoreType.DMA,
            pltpu.SemaphoreType.DMA,
        ],
        compiler_params=pltpu.CompilerParams(needs_layout_passes=False),
    )
    return run(x, phi_x)
